# baseline (speedup 1.0000x reference)
import functools

import jax
import jax.numpy as jnp
from jax import lax
from jax.experimental import pallas as pl
from jax.experimental.pallas import tpu as pltpu
from jax.experimental.pallas import tpu_sc as plsc

_B, _S = 4096, 50
_D = 32
_B_TOTAL = _B * _S
_NC, _NS = 2, 16
_NW = _NC * _NS
_B_PER_W = _B_TOTAL // _NW
_GW = 128
_N_G = _B_PER_W // _GW
_CHUNK = 10
_N_CHUNKS = _N_G // _CHUNK


@jax.jit
def _gather_rows(table, idx):
    mesh = plsc.VectorSubcoreMesh(core_axis_name="c", subcore_axis_name="s")

    @functools.partial(
        pl.kernel,
        mesh=mesh,
        out_type=jax.ShapeDtypeStruct((_B_TOTAL, _D), table.dtype),
        scratch_types=[
            pltpu.VMEM((_N_G, _GW), jnp.int32),
            pltpu.VMEM((_CHUNK * _GW, _D), table.dtype),
            pltpu.SemaphoreType.DMA,
        ],
        compiler_params=pltpu.CompilerParams(use_tc_tiling_on_sc=False),
    )
    def k(table_hbm, idx_hbm, out_hbm, idx_v, rows_v, sem):
        wid = lax.axis_index("s") * _NC + lax.axis_index("c")
        pltpu.sync_copy(idx_hbm.at[wid], idx_v)

        def chunk_body(c, carry):
            copies = [
                pltpu.async_copy(
                    table_hbm.at[idx_v.at[c * _CHUNK + j]],
                    rows_v.at[pl.ds(j * _GW, _GW)],
                    sem,
                )
                for j in range(_CHUNK)
            ]
            for cp in copies:
                cp.wait()
            pltpu.sync_copy(
                rows_v,
                out_hbm.at[pl.ds(wid * _B_PER_W + c * (_CHUNK * _GW), _CHUNK * _GW)],
            )
            return carry

        lax.fori_loop(0, _N_CHUNKS, chunk_body, 0)

    return k(table, idx)


def kernel(x, table):
    idx = x.reshape(_NW, _N_G, _GW)
    out = _gather_rows(table, idx)
    return out.reshape(_B, _S, _D)
